# Initial kernel scaffold; baseline (speedup 1.0000x reference)
import jax
import jax.numpy as jnp
from jax.experimental import pallas as pl


def kernel(offset, size, keypoint):
    raise NotImplementedError("write your pallas kernel here")


# speedup vs baseline: 46.6185x; 46.6185x over previous
import functools

import jax
import jax.numpy as jnp
from jax import lax
from jax.experimental import pallas as pl

_C = 80
_H = 128
_W = 128
_K = 10


def _postproc_kernel(off_ref, sz_ref, kp_ref, boxes_ref, cls_ref, sc_ref):
    x = kp_ref[...]
    ninf = jnp.float32(-jnp.inf)

    col = jnp.full((_C, 1, _W), ninf, dtype=jnp.float32)
    up = jnp.concatenate([x[:, 1:, :], col], axis=1)
    dn = jnp.concatenate([col, x[:, :-1, :]], axis=1)
    vy = jnp.maximum(x, jnp.maximum(up, dn))
    row = jnp.full((_C, _H, 1), ninf, dtype=jnp.float32)
    lf = jnp.concatenate([vy[:, :, 1:], row], axis=2)
    rt = jnp.concatenate([row, vy[:, :, :-1]], axis=2)
    pooled = jnp.maximum(vy, jnp.maximum(lf, rt))

    scores = jnp.where(pooled == x, x, jnp.float32(0.0))

    c_i = lax.broadcasted_iota(jnp.int32, (_C, _H, _W), 0)
    h_i = lax.broadcasted_iota(jnp.int32, (_C, _H, _W), 1)
    w_i = lax.broadcasted_iota(jnp.int32, (_C, _H, _W), 2)
    fidx = h_i * (_W * _C) + w_i * _C + c_i
    big = jnp.int32(2**31 - 1)

    hh = lax.broadcasted_iota(jnp.int32, (_H, _W), 0)
    ww = lax.broadcasted_iota(jnp.int32, (_H, _W), 1)
    off0m = off_ref[0]
    off1m = off_ref[1]
    sz0m = sz_ref[0]
    sz1m = sz_ref[1]

    vals = []
    clss = []
    rows = []
    for _ in range(_K):
        m = jnp.max(scores)
        idx = jnp.min(jnp.where(scores == m, fidx, big))
        scores = jnp.where(fidx == idx, -1.0, scores)

        sp = idx // _C
        cls = idx - sp * _C
        yi = sp // _W
        xi = sp - yi * _W
        y_f = yi.astype(jnp.float32)
        x_f = xi.astype(jnp.float32)

        sel = (hh == yi) & (ww == xi)
        o0 = jnp.sum(jnp.where(sel, off0m, 0.0))
        o1 = jnp.sum(jnp.where(sel, off1m, 0.0))
        s0 = jnp.sum(jnp.where(sel, sz0m, 0.0))
        s1 = jnp.sum(jnp.where(sel, sz1m, 0.0))

        pos0 = y_f + o1
        pos1 = x_f + o0
        hw0 = s1 * 0.5
        hw1 = s0 * 0.5
        lim = jnp.float32(_W - 1)
        b0 = jnp.clip(pos0 - hw0, 0.0, lim) * 4.0
        b1 = jnp.clip(pos1 - hw1, 0.0, lim) * 4.0
        b2 = jnp.clip(pos0 + hw0, 0.0, lim) * 4.0
        b3 = jnp.clip(pos1 + hw1, 0.0, lim) * 4.0

        vals.append(m)
        clss.append(cls)
        rows.append(jnp.stack([b0, b1, b2, b3]))

    boxes_ref[...] = jnp.stack(rows)
    cls_ref[...] = jnp.stack(clss)
    sc_ref[...] = jnp.stack(vals)


@jax.jit
def kernel(offset, size, keypoint):
    off = offset[0]
    sz = size[0]
    kp = keypoint[0]
    boxes, cls, sc = pl.pallas_call(
        _postproc_kernel,
        out_shape=(
            jax.ShapeDtypeStruct((_K, 4), jnp.float32),
            jax.ShapeDtypeStruct((_K,), jnp.int32),
            jax.ShapeDtypeStruct((_K,), jnp.float32),
        ),
    )(off, sz, kp)
    return boxes, cls, sc


# speedup vs baseline: 52.9422x; 1.1356x over previous
import jax
import jax.numpy as jnp
from jax import lax
from jax.experimental import pallas as pl

_C = 80
_H = 128
_W = 128
_K = 10


def _postproc_kernel(off_ref, sz_ref, kp_ref, boxes_ref, cls_ref, sc_ref):
    x = kp_ref[...]
    ninf = jnp.float32(-jnp.inf)

    plane = jnp.full((1, _W, _C), ninf, dtype=jnp.float32)
    up = jnp.concatenate([x[1:], plane], axis=0)
    dn = jnp.concatenate([plane, x[:-1]], axis=0)
    vy = jnp.maximum(x, jnp.maximum(up, dn))
    col = jnp.full((_H, 1, _C), ninf, dtype=jnp.float32)
    lf = jnp.concatenate([vy[:, 1:, :], col], axis=1)
    rt = jnp.concatenate([col, vy[:, :-1, :]], axis=1)
    pooled = jnp.maximum(vy, jnp.maximum(lf, rt))

    scores = jnp.where(pooled == x, x, jnp.float32(0.0))
    pmax = jnp.max(scores, axis=2)

    hh = lax.broadcasted_iota(jnp.int32, (_H, _W), 0)
    ww = lax.broadcasted_iota(jnp.int32, (_H, _W), 1)
    pidx = hh * _W + ww
    big = jnp.int32(2**31 - 1)

    wins = []
    for _ in range(_K):
        m = jnp.max(pmax)
        w = jnp.min(jnp.where(pmax == m, pidx, big))
        pmax = jnp.where(pidx == w, -1.0, pmax)
        wins.append(w)

    ninf_fib = jnp.full((_C,), ninf, dtype=jnp.float32)
    fibs = []
    fidxs = []
    for k in range(_K):
        w = wins[k]
        yi = w // _W
        xi = w - yi * _W
        mx = ninf_fib
        ctr = None
        for dy in (-1, 0, 1):
            for dx in (-1, 0, 1):
                yy = jnp.clip(yi + dy, 0, _H - 1)
                xx = jnp.clip(xi + dx, 0, _W - 1)
                v = kp_ref[pl.ds(yy, 1), pl.ds(xx, 1), :].reshape(_C)
                ok = ((yi + dy) >= 0) & ((yi + dy) < _H) & \
                     ((xi + dx) >= 0) & ((xi + dx) < _W)
                v = jnp.where(ok, v, ninf_fib)
                if dy == 0 and dx == 0:
                    ctr = v
                mx = jnp.maximum(mx, v)
        fib = jnp.where(mx == ctr, ctr, jnp.float32(0.0))
        fibs.append(fib)
        fidxs.append(w * _C + lax.iota(jnp.int32, _C))

    cand = jnp.stack(fibs)
    cidx = jnp.stack(fidxs)

    off0m = off_ref[0]
    off1m = off_ref[1]
    sz0m = sz_ref[0]
    sz1m = sz_ref[1]

    vals = []
    clss = []
    rows = []
    for _ in range(_K):
        m = jnp.max(cand)
        idx = jnp.min(jnp.where(cand == m, cidx, big))
        cand = jnp.where(cidx == idx, -1.0, cand)

        sp = idx // _C
        cls = idx - sp * _C
        yi = sp // _W
        xi = sp - yi * _W
        y_f = yi.astype(jnp.float32)
        x_f = xi.astype(jnp.float32)

        sel = (hh == yi) & (ww == xi)
        o0 = jnp.sum(jnp.where(sel, off0m, 0.0))
        o1 = jnp.sum(jnp.where(sel, off1m, 0.0))
        s0 = jnp.sum(jnp.where(sel, sz0m, 0.0))
        s1 = jnp.sum(jnp.where(sel, sz1m, 0.0))

        pos0 = y_f + o1
        pos1 = x_f + o0
        hw0 = s1 * 0.5
        hw1 = s0 * 0.5
        lim = jnp.float32(_W - 1)
        b0 = jnp.clip(pos0 - hw0, 0.0, lim) * 4.0
        b1 = jnp.clip(pos1 - hw1, 0.0, lim) * 4.0
        b2 = jnp.clip(pos0 + hw0, 0.0, lim) * 4.0
        b3 = jnp.clip(pos1 + hw1, 0.0, lim) * 4.0

        vals.append(m)
        clss.append(cls)
        rows.append(jnp.stack([b0, b1, b2, b3]))

    boxes_ref[...] = jnp.stack(rows)
    cls_ref[...] = jnp.stack(clss)
    sc_ref[...] = jnp.stack(vals)


@jax.jit
def kernel(offset, size, keypoint):
    off = offset[0]
    sz = size[0]
    kp = jnp.transpose(keypoint[0], (1, 2, 0))
    boxes, cls, sc = pl.pallas_call(
        _postproc_kernel,
        out_shape=(
            jax.ShapeDtypeStruct((_K, 4), jnp.float32),
            jax.ShapeDtypeStruct((_K,), jnp.int32),
            jax.ShapeDtypeStruct((_K,), jnp.float32),
        ),
    )(off, sz, kp)
    return boxes, cls, sc


# speedup vs baseline: 99.1869x; 1.8735x over previous
import jax
import jax.numpy as jnp
from jax import lax
from jax.experimental import pallas as pl

_C = 80
_H = 128
_W = 128
_K = 10


def _row_slab_scores(kp_ref, yi):
    ninf = jnp.float32(-jnp.inf)
    slabs = []
    for dy in (-1, 0, 1):
        yy = jnp.clip(yi + dy, 0, _H - 1)
        s = kp_ref[:, pl.ds(yy, 1), :].reshape(_C, _W)
        ok = ((yi + dy) >= 0) & ((yi + dy) < _H)
        s = jnp.where(ok, s, ninf)
        slabs.append(s)
    ctr = slabs[1]
    vmax = jnp.maximum(slabs[0], jnp.maximum(ctr, slabs[2]))
    col = jnp.full((_C, 1), ninf, dtype=jnp.float32)
    lf = jnp.concatenate([vmax[:, 1:], col], axis=1)
    rt = jnp.concatenate([col, vmax[:, :-1]], axis=1)
    pooled = jnp.maximum(vmax, jnp.maximum(lf, rt))
    return jnp.where(pooled == ctr, ctr, jnp.float32(0.0))


def _postproc_kernel(off_ref, sz_ref, kp_ref, boxes_ref, cls_ref, sc_ref):
    x = kp_ref[...]
    ninf = jnp.float32(-jnp.inf)

    plane = jnp.full((_C, 1, _W), ninf, dtype=jnp.float32)
    up = jnp.concatenate([x[:, 1:, :], plane], axis=1)
    dn = jnp.concatenate([plane, x[:, :-1, :]], axis=1)
    vy = jnp.maximum(x, jnp.maximum(up, dn))
    col = jnp.full((_C, _H, 1), ninf, dtype=jnp.float32)
    lf = jnp.concatenate([vy[:, :, 1:], col], axis=2)
    rt = jnp.concatenate([col, vy[:, :, :-1]], axis=2)
    pooled = jnp.maximum(vy, jnp.maximum(lf, rt))

    scores = jnp.where(pooled == x, x, jnp.float32(0.0))
    pmax = jnp.max(scores, axis=0)

    hh = lax.broadcasted_iota(jnp.int32, (_H, _W), 0)
    ww = lax.broadcasted_iota(jnp.int32, (_H, _W), 1)
    pidx = hh * _W + ww
    big = jnp.int32(2**31 - 1)

    wins = []
    for _ in range(_K):
        m = jnp.max(pmax)
        w = jnp.min(jnp.where(pmax == m, pidx, big))
        pmax = jnp.where(pidx == w, -1.0, pmax)
        wins.append(w)

    lane_w = lax.broadcasted_iota(jnp.int32, (_C, _W), 1)
    lane1 = lax.broadcasted_iota(jnp.int32, (1, _W), 1)

    fibs = []
    fidxs = []
    decode_rows = []
    for k in range(_K):
        w = wins[k]
        yi = w // _W
        xi = w - yi * _W

        slab = _row_slab_scores(kp_ref, yi)
        fib = jnp.sum(jnp.where(lane_w == xi, slab, 0.0), axis=1)
        fibs.append(fib)
        fidxs.append(w * _C + lax.iota(jnp.int32, _C))

        sel = lane1 == xi
        o0 = jnp.sum(jnp.where(sel, off_ref[0, pl.ds(yi, 1), :], 0.0))
        o1 = jnp.sum(jnp.where(sel, off_ref[1, pl.ds(yi, 1), :], 0.0))
        s0 = jnp.sum(jnp.where(sel, sz_ref[0, pl.ds(yi, 1), :], 0.0))
        s1 = jnp.sum(jnp.where(sel, sz_ref[1, pl.ds(yi, 1), :], 0.0))
        decode_rows.append(jnp.stack([o0, o1, s0, s1]))

    cand = jnp.stack(fibs)
    cidx = jnp.stack(fidxs)
    dec = jnp.stack(decode_rows)
    winv = jnp.stack(wins)

    vals = []
    clss = []
    rows = []
    for _ in range(_K):
        m = jnp.max(cand)
        idx = jnp.min(jnp.where(cand == m, cidx, big))
        cand = jnp.where(cidx == idx, -1.0, cand)

        sp = idx // _C
        cls = idx - sp * _C
        yi = sp // _W
        xi = sp - yi * _W
        y_f = yi.astype(jnp.float32)
        x_f = xi.astype(jnp.float32)

        psel = winv == sp
        o0 = jnp.sum(jnp.where(psel, dec[:, 0], 0.0))
        o1 = jnp.sum(jnp.where(psel, dec[:, 1], 0.0))
        s0 = jnp.sum(jnp.where(psel, dec[:, 2], 0.0))
        s1 = jnp.sum(jnp.where(psel, dec[:, 3], 0.0))

        pos0 = y_f + o1
        pos1 = x_f + o0
        hw0 = s1 * 0.5
        hw1 = s0 * 0.5
        lim = jnp.float32(_W - 1)
        b0 = jnp.clip(pos0 - hw0, 0.0, lim) * 4.0
        b1 = jnp.clip(pos1 - hw1, 0.0, lim) * 4.0
        b2 = jnp.clip(pos0 + hw0, 0.0, lim) * 4.0
        b3 = jnp.clip(pos1 + hw1, 0.0, lim) * 4.0

        vals.append(m)
        clss.append(cls)
        rows.append(jnp.stack([b0, b1, b2, b3]))

    boxes_ref[...] = jnp.stack(rows)
    cls_ref[...] = jnp.stack(clss)
    sc_ref[...] = jnp.stack(vals)


@jax.jit
def kernel(offset, size, keypoint):
    off = offset[0]
    sz = size[0]
    kp = keypoint[0]
    boxes, cls, sc = pl.pallas_call(
        _postproc_kernel,
        out_shape=(
            jax.ShapeDtypeStruct((_K, 4), jnp.float32),
            jax.ShapeDtypeStruct((_K,), jnp.int32),
            jax.ShapeDtypeStruct((_K,), jnp.float32),
        ),
    )(off, sz, kp)
    return boxes, cls, sc


# speedup vs baseline: 106.8725x; 1.0775x over previous
import jax
import jax.numpy as jnp
from jax import lax
from jax.experimental import pallas as pl
from jax.experimental.pallas import tpu as pltpu

_C = 80
_H = 128
_W = 128
_K = 10


def _postproc_kernel(off_ref, sz_ref, kp_ref, boxes_ref, cls_ref, sc_ref,
                     scores_ref):
    ninf = jnp.float32(-jnp.inf)
    row = jnp.full((1, _W), ninf, dtype=jnp.float32)
    colv = jnp.full((_H, 1), ninf, dtype=jnp.float32)

    pmax = jnp.zeros((_H, _W), dtype=jnp.float32)
    for c in range(_C):
        xc = kp_ref[c]
        up = jnp.concatenate([xc[1:], row], axis=0)
        dn = jnp.concatenate([row, xc[:-1]], axis=0)
        vy = jnp.maximum(xc, jnp.maximum(up, dn))
        lf = jnp.concatenate([vy[:, 1:], colv], axis=1)
        rt = jnp.concatenate([colv, vy[:, :-1]], axis=1)
        pooled = jnp.maximum(vy, jnp.maximum(lf, rt))
        sc_c = jnp.where(pooled == xc, xc, jnp.float32(0.0))
        scores_ref[c] = sc_c
        pmax = jnp.maximum(pmax, sc_c)

    hh = lax.broadcasted_iota(jnp.int32, (_H, _W), 0)
    ww = lax.broadcasted_iota(jnp.int32, (_H, _W), 1)
    pidx = hh * _W + ww
    big = jnp.int32(2**31 - 1)

    wins = []
    for _ in range(_K):
        m = jnp.max(pmax)
        w = jnp.min(jnp.where(pmax == m, pidx, big))
        pmax = jnp.where(pidx == w, -1.0, pmax)
        wins.append(w)

    lane_w = lax.broadcasted_iota(jnp.int32, (_C, _W), 1)
    lane1 = lax.broadcasted_iota(jnp.int32, (1, _W), 1)

    fibs = []
    fidxs = []
    decode_rows = []
    for k in range(_K):
        w = wins[k]
        yi = w // _W
        xi = w - yi * _W

        slab = scores_ref[:, pl.ds(yi, 1), :].reshape(_C, _W)
        fib = jnp.sum(jnp.where(lane_w == xi, slab, 0.0), axis=1)
        fibs.append(fib)
        fidxs.append(w * _C + lax.iota(jnp.int32, _C))

        sel = lane1 == xi
        o0 = jnp.sum(jnp.where(sel, off_ref[0, pl.ds(yi, 1), :], 0.0))
        o1 = jnp.sum(jnp.where(sel, off_ref[1, pl.ds(yi, 1), :], 0.0))
        s0 = jnp.sum(jnp.where(sel, sz_ref[0, pl.ds(yi, 1), :], 0.0))
        s1 = jnp.sum(jnp.where(sel, sz_ref[1, pl.ds(yi, 1), :], 0.0))
        decode_rows.append(jnp.stack([o0, o1, s0, s1]))

    cand = jnp.stack(fibs)
    cidx = jnp.stack(fidxs)
    dec = jnp.stack(decode_rows)
    winv = jnp.stack(wins)

    vals = []
    clss = []
    rows = []
    for _ in range(_K):
        m = jnp.max(cand)
        idx = jnp.min(jnp.where(cand == m, cidx, big))
        cand = jnp.where(cidx == idx, -1.0, cand)

        sp = idx // _C
        cls = idx - sp * _C
        yi = sp // _W
        xi = sp - yi * _W
        y_f = yi.astype(jnp.float32)
        x_f = xi.astype(jnp.float32)

        psel = winv == sp
        o0 = jnp.sum(jnp.where(psel, dec[:, 0], 0.0))
        o1 = jnp.sum(jnp.where(psel, dec[:, 1], 0.0))
        s0 = jnp.sum(jnp.where(psel, dec[:, 2], 0.0))
        s1 = jnp.sum(jnp.where(psel, dec[:, 3], 0.0))

        pos0 = y_f + o1
        pos1 = x_f + o0
        hw0 = s1 * 0.5
        hw1 = s0 * 0.5
        lim = jnp.float32(_W - 1)
        b0 = jnp.clip(pos0 - hw0, 0.0, lim) * 4.0
        b1 = jnp.clip(pos1 - hw1, 0.0, lim) * 4.0
        b2 = jnp.clip(pos0 + hw0, 0.0, lim) * 4.0
        b3 = jnp.clip(pos1 + hw1, 0.0, lim) * 4.0

        vals.append(m)
        clss.append(cls)
        rows.append(jnp.stack([b0, b1, b2, b3]))

    boxes_ref[...] = jnp.stack(rows)
    cls_ref[...] = jnp.stack(clss)
    sc_ref[...] = jnp.stack(vals)


@jax.jit
def kernel(offset, size, keypoint):
    off = offset[0]
    sz = size[0]
    kp = keypoint[0]
    boxes, cls, sc = pl.pallas_call(
        _postproc_kernel,
        out_shape=(
            jax.ShapeDtypeStruct((_K, 4), jnp.float32),
            jax.ShapeDtypeStruct((_K,), jnp.int32),
            jax.ShapeDtypeStruct((_K,), jnp.float32),
        ),
        scratch_shapes=[pltpu.VMEM((_C, _H, _W), jnp.float32)],
    )(off, sz, kp)
    return boxes, cls, sc
